# baseline (speedup 1.0000x reference)
import functools

import numpy as np

import jax
import jax.numpy as jnp
from jax import lax
from jax.experimental import pallas as pl
from jax.experimental.pallas import tpu as pltpu
from jax.experimental.pallas import tpu_sc as plsc

N_NODES = 10000
N_EDGES = 320000
IN_CH = 128
HID = 64
OUT_CH = 5
N_IDX = 2048

NC = 2
NS = 16
NW = NC * NS
CH = 128
NP = 10240
NPH = NP // 2
ROWS_PER_TILE = NP // NS
NCH = N_EDGES // CH
CPW = 80
CPW_LAST = NCH - (NW - 1) * CPW
CPT = 160
CPT_LAST = NCH - (NS - 1) * CPT
DEGW = 16
NBUF = 5
DEG_DEPTH = 8
SEL_PER_TILE = N_IDX // NS
DIS_ROWS = NP // NW

_mesh = plsc.VectorSubcoreMesh(core_axis_name="c", subcore_axis_name="s")
_f32 = jnp.float32
_sc_params = pltpu.CompilerParams(use_tc_tiling_on_sc=False)
_sc_params_nl = pltpu.CompilerParams(use_tc_tiling_on_sc=False,
                                     needs_layout_passes=False)

_ZEROS_DEG = np.zeros((ROWS_PER_TILE, DEGW), np.float32)
_ONES_DEG = np.ones((CH, DEGW), np.float32)
_ZEROS_HID = np.zeros((ROWS_PER_TILE, HID), np.float32)


def _load_chunks(hbm3, idx_v, start, n_main, n_last, is_last):
    @pl.when(jnp.logical_not(is_last))
    def _():
        pltpu.sync_copy(hbm3.at[pl.ds(start, n_main)], idx_v)

    @pl.when(is_last)
    def _():
        pltpu.sync_copy(hbm3.at[pl.ds(NCH - n_last, n_last)],
                        idx_v.at[pl.ds(0, n_last)])

    return lax.select(is_last, n_last, n_main)


@functools.partial(
    pl.kernel,
    out_type=(
        jax.ShapeDtypeStruct((NP, HID), _f32),
        jax.ShapeDtypeStruct((NP, HID), _f32),
    ),
    mesh=_mesh,
    compiler_params=_sc_params_nl,
    scratch_types=[
        pltpu.VMEM((CPT, CH), jnp.int32),
        pltpu.VMEM((CH, DEGW), _f32),
        pltpu.VMEM((DIS_ROWS, DEGW), _f32),
        pltpu.VMEM((DIS_ROWS, HID), _f32),
        pltpu.VMEM((DIS_ROWS, HID), _f32),
        pltpu.VMEM((DIS_ROWS, HID), _f32),
        pltpu.VMEM_SHARED((NP, DEGW), _f32),
        pltpu.SemaphoreType.DMA,
        pltpu.SemaphoreType.DMA,
    ],
)
def _dis_kernel(dst_hbm, h0_hbm, ones_hbm, zeros_hbm, dis_hbm, ht0_hbm,
                dst_v, ones_v, deg_v, dis_v, h0_v, ht0_v, acc_sp, sem, sem_h):
    c = lax.axis_index("c")
    s = lax.axis_index("s")
    wid = c * NS + s
    drow0 = wid * DIS_ROWS
    pltpu.async_copy(h0_hbm.at[pl.ds(drow0, DIS_ROWS)], h0_v, sem_h)
    pltpu.sync_copy(ones_hbm, ones_v)
    n_real = _load_chunks(dst_hbm, dst_v, s * CPT, CPT, CPT_LAST, s == NS - 1)
    row0 = s * ROWS_PER_TILE
    pltpu.sync_copy(zeros_hbm, acc_sp.at[pl.ds(row0, ROWS_PER_TILE)])
    plsc.subcore_barrier()

    for j in range(DEG_DEPTH):
        pltpu.async_copy(ones_v, acc_sp.at[dst_v.at[j]], sem, add=True)

    def body(k, carry):
        pltpu.make_async_copy(ones_v, acc_sp.at[dst_v.at[k]], sem).wait()
        pltpu.async_copy(ones_v, acc_sp.at[dst_v.at[k + DEG_DEPTH]], sem,
                         add=True)
        return carry

    lax.fori_loop(0, n_real - DEG_DEPTH, body, 0)
    for j in range(DEG_DEPTH):
        pltpu.make_async_copy(ones_v, acc_sp.at[dst_v.at[j]], sem).wait()

    plsc.subcore_barrier()
    pltpu.sync_copy(acc_sp.at[pl.ds(drow0, DIS_ROWS)], deg_v)
    pltpu.make_async_copy(h0_hbm.at[pl.ds(drow0, DIS_ROWS)], h0_v,
                          sem_h).wait()

    def dis_body(r, carry):
        deg = deg_v[r] + 1.0
        i = plsc.bitcast(deg, jnp.int32)
        y = plsc.bitcast(0x5F3759DF - lax.shift_right_logical(i, 1), _f32)
        y = y * (1.5 - 0.5 * deg * y * y)
        y = y * (1.5 - 0.5 * deg * y * y)
        y = y * (1.5 - 0.5 * deg * y * y)
        for k in range(HID // DEGW):
            dis_v[r, pl.ds(k * DEGW, DEGW)] = y
            ht0_v[r, pl.ds(k * DEGW, DEGW)] = h0_v[r, pl.ds(k * DEGW, DEGW)] * y
        return carry

    lax.fori_loop(0, DIS_ROWS, dis_body, 0)
    pltpu.sync_copy(dis_v, dis_hbm.at[pl.ds(drow0, DIS_ROWS)])
    pltpu.sync_copy(ht0_v, ht0_hbm.at[pl.ds(drow0, DIS_ROWS)])


def _agg_prologue(ht_hbm, src_hbm, dst_hbm, zeros_hbm,
                  src_v, dst_v, rows, gsems, ssems, acc_sp):
    c = lax.axis_index("c")
    s = lax.axis_index("s")
    wid = c * NS + s
    last = wid == NW - 1
    n_real = _load_chunks(src_hbm, src_v, wid * CPW, CPW, CPW_LAST, last)
    _load_chunks(dst_hbm, dst_v, wid * CPW, CPW, CPW_LAST, last)
    row0 = s * ROWS_PER_TILE
    pltpu.sync_copy(zeros_hbm, acc_sp.at[pl.ds(row0, ROWS_PER_TILE)])
    plsc.subcore_barrier()

    for b in range(NBUF):
        pltpu.async_copy(ht_hbm.at[src_v.at[b]], rows[b], gsems[b])

    def body(k, carry):
        j0 = NBUF * k
        for b in range(NBUF):
            pltpu.make_async_copy(ht_hbm.at[src_v.at[j0 + b]],
                                  rows[b], gsems[b]).wait()
            pltpu.async_copy(rows[b], acc_sp.at[dst_v.at[j0 + b]],
                             ssems[b], add=True)
        for b in range(NBUF):
            jn = j0 + NBUF + b

            @pl.when(jn < n_real)
            def _(b=b, jn=jn):
                pltpu.make_async_copy(rows[b], acc_sp.at[dst_v.at[jn]],
                                      ssems[b]).wait()
                pltpu.async_copy(ht_hbm.at[src_v.at[jn]], rows[b], gsems[b])

        return carry

    lax.fori_loop(0, n_real // NBUF, body, 0)
    for b in range(NBUF):
        pltpu.make_async_copy(rows[b], acc_sp.at[dst_v.at[b]], ssems[b]).wait()

    plsc.subcore_barrier()
    return c, s, row0


@functools.partial(
    pl.kernel,
    out_type=jax.ShapeDtypeStruct((NC, NP, HID), _f32),
    mesh=_mesh,
    compiler_params=_sc_params,
    scratch_types=[
        pltpu.VMEM((CPW, CH), jnp.int32),
        pltpu.VMEM((CPW, CH), jnp.int32),
    ] + [pltpu.VMEM((CH, HID), _f32)] * NBUF
      + [pltpu.SemaphoreType.DMA] * (2 * NBUF)
      + [pltpu.VMEM_SHARED((NP, HID), _f32)],
)
def _agg_kernel(ht_hbm, src_hbm, dst_hbm, zeros_hbm, out_hbm,
                src_v, dst_v, *bufs):
    rows = bufs[:NBUF]
    gsems = bufs[NBUF:2 * NBUF]
    ssems = bufs[2 * NBUF:3 * NBUF]
    acc_sp = bufs[3 * NBUF]
    c, s, row0 = _agg_prologue(ht_hbm, src_hbm, dst_hbm, zeros_hbm,
                               src_v, dst_v, rows, gsems, ssems, acc_sp)
    pltpu.sync_copy(acc_sp.at[pl.ds(row0, ROWS_PER_TILE)],
                    out_hbm.at[c].at[pl.ds(row0, ROWS_PER_TILE)])


@functools.partial(
    pl.kernel,
    out_type=(
        jax.ShapeDtypeStruct((NC, NP, HID), _f32),
        jax.ShapeDtypeStruct((NC, N_IDX, HID), _f32),
        jax.ShapeDtypeStruct((N_IDX, HID), _f32),
        jax.ShapeDtypeStruct((N_IDX, HID), _f32),
    ),
    mesh=_mesh,
    compiler_params=_sc_params,
    scratch_types=[
        pltpu.VMEM((CPW, CH), jnp.int32),
        pltpu.VMEM((CPW, CH), jnp.int32),
    ] + [pltpu.VMEM((CH, HID), _f32)] * NBUF
      + [pltpu.SemaphoreType.DMA] * (2 * NBUF)
      + [
        pltpu.VMEM((SEL_PER_TILE,), jnp.int32),
        pltpu.VMEM((SEL_PER_TILE, HID), _f32),
        pltpu.VMEM_SHARED((NP, HID), _f32),
    ],
)
def _agg_sel_kernel(ht_hbm, src_hbm, dst_hbm, zeros_hbm, dis_hbm, idx_hbm,
                    out_hbm, selacc_hbm, selht_hbm, seldis_hbm,
                    src_v, dst_v, *bufs):
    rows = bufs[:NBUF]
    gsems = bufs[NBUF:2 * NBUF]
    ssems = bufs[2 * NBUF:3 * NBUF]
    idxsel_v, selrow_v, acc_sp = bufs[3 * NBUF:]

    c, s, row0 = _agg_prologue(ht_hbm, src_hbm, dst_hbm, zeros_hbm,
                               src_v, dst_v, rows, gsems, ssems, acc_sp)
    pltpu.sync_copy(acc_sp.at[pl.ds(row0, ROWS_PER_TILE)],
                    out_hbm.at[c].at[pl.ds(row0, ROWS_PER_TILE)])

    base = s * SEL_PER_TILE
    pltpu.sync_copy(idx_hbm.at[pl.ds(base, SEL_PER_TILE)], idxsel_v)

    @pl.when(c == 0)
    def _():
        pltpu.async_copy(ht_hbm.at[idxsel_v], selrow_v, gsems[0]).wait()
        pltpu.sync_copy(selrow_v, selht_hbm.at[pl.ds(base, SEL_PER_TILE)])

    @pl.when(c == 1)
    def _():
        pltpu.async_copy(dis_hbm.at[idxsel_v], selrow_v, gsems[0]).wait()
        pltpu.sync_copy(selrow_v, seldis_hbm.at[pl.ds(base, SEL_PER_TILE)])

    plsc.subcore_barrier()
    pltpu.async_copy(out_hbm.at[c].at[idxsel_v], selrow_v, gsems[1]).wait()
    pltpu.sync_copy(selrow_v,
                    selacc_hbm.at[c].at[pl.ds(base, SEL_PER_TILE)])


MBLK = 1000


def _tcMM_body(x_ref, w0_ref, h_ref):
    h_ref[...] = jnp.dot(x_ref[...], w0_ref[...], preferred_element_type=_f32)


_tcMM = pl.pallas_call(
    _tcMM_body,
    grid=(N_NODES // (2 * MBLK),),
    in_specs=[
        pl.BlockSpec((MBLK, 2 * IN_CH), lambda i: (i, 0)),
        pl.BlockSpec((2 * IN_CH, 2 * HID), lambda i: (0, 0)),
    ],
    out_specs=pl.BlockSpec((MBLK, 2 * HID), lambda i: (i, 0)),
    out_shape=jax.ShapeDtypeStruct((NPH, 2 * HID), _f32),
)


def _tcB_body(accp_ref, ht0_ref, dis_ref, w1_ref, b0_ref, ht1_ref):
    dis = dis_ref[...]
    pre = (accp_ref[0] + accp_ref[1] + ht0_ref[...]) * dis + b0_ref[...]
    h1 = jnp.where(pre >= 0, pre, 0.01 * pre)
    ht1_ref[...] = jnp.dot(h1, w1_ref[...], preferred_element_type=_f32) * dis


_tcB = pl.pallas_call(
    _tcB_body,
    grid=(N_NODES // (2 * MBLK),),
    in_specs=[
        pl.BlockSpec((NC, MBLK, 2 * HID), lambda i: (0, i, 0)),
        pl.BlockSpec((MBLK, 2 * HID), lambda i: (i, 0)),
        pl.BlockSpec((MBLK, 2 * HID), lambda i: (i, 0)),
        pl.BlockSpec((2 * HID, 2 * HID), lambda i: (0, 0)),
        pl.BlockSpec((1, 2 * HID), lambda i: (0, 0)),
    ],
    out_specs=pl.BlockSpec((MBLK, 2 * HID), lambda i: (i, 0)),
    out_shape=jax.ShapeDtypeStruct((NPH, 2 * HID), _f32),
)


def _tcF_body(sacc_ref, sh_ref, sd_ref, b1_ref, wm_ref, bm_ref,
              hsel_ref, out_ref):
    pre = (sacc_ref[0] + sacc_ref[1] + sh_ref[...]) * sd_ref[...] + b1_ref[...]
    hsel = jnp.where(pre >= 0, pre, 0.01 * pre)
    hsel_ref[...] = hsel
    z = jnp.dot(hsel, wm_ref[...], preferred_element_type=_f32) + bm_ref[...]
    out_ref[...] = 1.0 / (1.0 + jnp.exp(-z))


_tcF = pl.pallas_call(
    _tcF_body,
    out_shape=[
        jax.ShapeDtypeStruct((N_IDX // 2, 2 * HID), _f32),
        jax.ShapeDtypeStruct((N_IDX // 2, 256), _f32),
    ],
)


def _blockdiag(w, rows, cols):
    z = jnp.zeros((2 * rows, 2 * cols), _f32)
    return z.at[:rows, :cols].set(w).at[rows:, cols:].set(w)


def kernel(x, edge_index, idx, W0, b0, W1, b1, Wm, bm):
    x_p = x.reshape(N_NODES // 2, 2 * IN_CH)

    w0bd = _blockdiag(W0, IN_CH, HID)
    w1bd = _blockdiag(W1, HID, HID)
    wm_pad = jnp.pad(Wm, ((0, 0), (0, 128 - OUT_CH)))
    wmbd = _blockdiag(wm_pad, HID, 128)
    b0p = jnp.concatenate([b0, b0]).reshape(1, 2 * HID)
    b1p = jnp.concatenate([b1, b1]).reshape(1, 2 * HID)
    bm_pad = jnp.pad(bm, (0, 128 - OUT_CH))
    bmp = jnp.concatenate([bm_pad, bm_pad]).reshape(1, 256)

    h0_p = _tcMM(x_p, w0bd)
    ei_late, _ = lax.optimization_barrier((edge_index, h0_p))
    dst2d = ei_late[1].reshape(NCH, CH)
    src2d = ei_late[0].reshape(NCH, CH)
    dis64, ht0 = _dis_kernel(dst2d, h0_p.reshape(NP, HID),
                             _ONES_DEG, _ZEROS_DEG)
    dis_p = dis64.reshape(NPH, 2 * HID)
    ht0_p = ht0.reshape(NPH, 2 * HID)
    acc0 = _agg_kernel(ht0, src2d, dst2d, _ZEROS_HID)
    ht1_p = _tcB(acc0.reshape(NC, NPH, 2 * HID), ht0_p, dis_p, w1bd, b0p)
    _, sel_acc, sel_ht, sel_dis = _agg_sel_kernel(
        ht1_p.reshape(NP, HID), src2d, dst2d, _ZEROS_HID, dis64, idx)
    hsel_p, out_pp = _tcF(sel_acc.reshape(NC, N_IDX // 2, 2 * HID),
                          sel_ht.reshape(N_IDX // 2, 2 * HID),
                          sel_dis.reshape(N_IDX // 2, 2 * HID),
                          b1p, wmbd, bmp)
    h_sel = hsel_p.reshape(N_IDX, HID)
    out = out_pp.reshape(N_IDX, 128)[:, :OUT_CH]
    return (h_sel, out)


# speedup vs baseline: 1.0478x; 1.0478x over previous
import functools

import numpy as np

import jax
import jax.numpy as jnp
from jax import lax
from jax.experimental import pallas as pl
from jax.experimental.pallas import tpu as pltpu
from jax.experimental.pallas import tpu_sc as plsc

N_NODES = 10000
N_EDGES = 320000
IN_CH = 128
HID = 64
OUT_CH = 5
N_IDX = 2048

NC = 2
NS = 16
NW = NC * NS
CH = 128
NP = 10240
NPH = NP // 2
ROWS_PER_TILE = NP // NS
NCH = N_EDGES // CH
CPW = 80
CPW_LAST = NCH - (NW - 1) * CPW
CPT = 160
CPT_LAST = NCH - (NS - 1) * CPT
DEGW = 16
NBUF = 5
DEG_DEPTH = 8
SEL_PER_TILE = N_IDX // NS
DIS_ROWS = NP // NW

_mesh = plsc.VectorSubcoreMesh(core_axis_name="c", subcore_axis_name="s")
_f32 = jnp.float32
_sc_params = pltpu.CompilerParams(use_tc_tiling_on_sc=False)
_sc_params_nl = pltpu.CompilerParams(use_tc_tiling_on_sc=False,
                                     needs_layout_passes=False)

_ZEROS_DEG = np.zeros((ROWS_PER_TILE, DEGW), np.float32)
_ONES_DEG = np.ones((CH, DEGW), np.float32)
_ZEROS_HID = np.zeros((ROWS_PER_TILE, HID), np.float32)


def _load_chunks(hbm3, idx_v, start, n_main, n_last, is_last):
    @pl.when(jnp.logical_not(is_last))
    def _():
        pltpu.sync_copy(hbm3.at[pl.ds(start, n_main)], idx_v)

    @pl.when(is_last)
    def _():
        pltpu.sync_copy(hbm3.at[pl.ds(NCH - n_last, n_last)],
                        idx_v.at[pl.ds(0, n_last)])

    return lax.select(is_last, n_last, n_main)


@functools.partial(
    pl.kernel,
    out_type=jax.ShapeDtypeStruct((NP, HID), _f32),
    mesh=_mesh,
    compiler_params=_sc_params_nl,
    scratch_types=[
        pltpu.VMEM((CPT, CH), jnp.int32),
        pltpu.VMEM((CH, DEGW), _f32),
        pltpu.VMEM((DIS_ROWS, DEGW), _f32),
        pltpu.VMEM((DIS_ROWS, HID), _f32),
        pltpu.VMEM_SHARED((NP, DEGW), _f32),
        pltpu.SemaphoreType.DMA,
    ],
)
def _dis_kernel(dst_hbm, ones_hbm, zeros_hbm, dis_hbm,
                dst_v, ones_v, deg_v, dis_v, acc_sp, sem):
    c = lax.axis_index("c")
    s = lax.axis_index("s")
    pltpu.sync_copy(ones_hbm, ones_v)
    n_real = _load_chunks(dst_hbm, dst_v, s * CPT, CPT, CPT_LAST, s == NS - 1)
    row0 = s * ROWS_PER_TILE
    pltpu.sync_copy(zeros_hbm, acc_sp.at[pl.ds(row0, ROWS_PER_TILE)])
    plsc.subcore_barrier()

    for j in range(DEG_DEPTH):
        pltpu.async_copy(ones_v, acc_sp.at[dst_v.at[j]], sem, add=True)

    def body(k, carry):
        pltpu.make_async_copy(ones_v, acc_sp.at[dst_v.at[k]], sem).wait()
        pltpu.async_copy(ones_v, acc_sp.at[dst_v.at[k + DEG_DEPTH]], sem,
                         add=True)
        return carry

    lax.fori_loop(0, n_real - DEG_DEPTH, body, 0)
    for j in range(DEG_DEPTH):
        pltpu.make_async_copy(ones_v, acc_sp.at[dst_v.at[j]], sem).wait()

    plsc.subcore_barrier()
    wid = c * NS + s
    drow0 = wid * DIS_ROWS
    pltpu.sync_copy(acc_sp.at[pl.ds(drow0, DIS_ROWS)], deg_v)

    def dis_body(r, carry):
        deg = deg_v[r] + 1.0
        i = plsc.bitcast(deg, jnp.int32)
        y = plsc.bitcast(0x5F3759DF - lax.shift_right_logical(i, 1), _f32)
        y = y * (1.5 - 0.5 * deg * y * y)
        y = y * (1.5 - 0.5 * deg * y * y)
        y = y * (1.5 - 0.5 * deg * y * y)
        for k in range(HID // DEGW):
            dis_v[r, pl.ds(k * DEGW, DEGW)] = y
        return carry

    lax.fori_loop(0, DIS_ROWS, dis_body, 0)
    pltpu.sync_copy(dis_v, dis_hbm.at[pl.ds(drow0, DIS_ROWS)])


def _agg_prologue(ht_hbm, src_hbm, dst_hbm, zeros_hbm,
                  src_v, dst_v, rows, gsems, ssems, acc_sp):
    c = lax.axis_index("c")
    s = lax.axis_index("s")
    wid = c * NS + s
    last = wid == NW - 1
    n_real = _load_chunks(src_hbm, src_v, wid * CPW, CPW, CPW_LAST, last)
    _load_chunks(dst_hbm, dst_v, wid * CPW, CPW, CPW_LAST, last)
    row0 = s * ROWS_PER_TILE
    pltpu.sync_copy(zeros_hbm, acc_sp.at[pl.ds(row0, ROWS_PER_TILE)])
    plsc.subcore_barrier()

    for b in range(NBUF):
        pltpu.async_copy(ht_hbm.at[src_v.at[b]], rows[b], gsems[b])

    def body(k, carry):
        j0 = NBUF * k
        for b in range(NBUF):
            pltpu.make_async_copy(ht_hbm.at[src_v.at[j0 + b]],
                                  rows[b], gsems[b]).wait()
            pltpu.async_copy(rows[b], acc_sp.at[dst_v.at[j0 + b]],
                             ssems[b], add=True)
        for b in range(NBUF):
            jn = j0 + NBUF + b

            @pl.when(jn < n_real)
            def _(b=b, jn=jn):
                pltpu.make_async_copy(rows[b], acc_sp.at[dst_v.at[jn]],
                                      ssems[b]).wait()
                pltpu.async_copy(ht_hbm.at[src_v.at[jn]], rows[b], gsems[b])

        return carry

    lax.fori_loop(0, n_real // NBUF, body, 0)
    for b in range(NBUF):
        pltpu.make_async_copy(rows[b], acc_sp.at[dst_v.at[b]], ssems[b]).wait()

    plsc.subcore_barrier()
    return c, s, row0


@functools.partial(
    pl.kernel,
    out_type=jax.ShapeDtypeStruct((NC, NP, HID), _f32),
    mesh=_mesh,
    compiler_params=_sc_params,
    scratch_types=[
        pltpu.VMEM((CPW, CH), jnp.int32),
        pltpu.VMEM((CPW, CH), jnp.int32),
    ] + [pltpu.VMEM((CH, HID), _f32)] * NBUF
      + [pltpu.SemaphoreType.DMA] * (2 * NBUF)
      + [pltpu.VMEM_SHARED((NP, HID), _f32)],
)
def _agg_kernel(ht_hbm, src_hbm, dst_hbm, zeros_hbm, out_hbm,
                src_v, dst_v, *bufs):
    rows = bufs[:NBUF]
    gsems = bufs[NBUF:2 * NBUF]
    ssems = bufs[2 * NBUF:3 * NBUF]
    acc_sp = bufs[3 * NBUF]
    c, s, row0 = _agg_prologue(ht_hbm, src_hbm, dst_hbm, zeros_hbm,
                               src_v, dst_v, rows, gsems, ssems, acc_sp)
    pltpu.sync_copy(acc_sp.at[pl.ds(row0, ROWS_PER_TILE)],
                    out_hbm.at[c].at[pl.ds(row0, ROWS_PER_TILE)])


@functools.partial(
    pl.kernel,
    out_type=(
        jax.ShapeDtypeStruct((NC, NP, HID), _f32),
        jax.ShapeDtypeStruct((NC, N_IDX, HID), _f32),
        jax.ShapeDtypeStruct((N_IDX, HID), _f32),
        jax.ShapeDtypeStruct((N_IDX, HID), _f32),
    ),
    mesh=_mesh,
    compiler_params=_sc_params,
    scratch_types=[
        pltpu.VMEM((CPW, CH), jnp.int32),
        pltpu.VMEM((CPW, CH), jnp.int32),
    ] + [pltpu.VMEM((CH, HID), _f32)] * NBUF
      + [pltpu.SemaphoreType.DMA] * (2 * NBUF)
      + [
        pltpu.VMEM((SEL_PER_TILE,), jnp.int32),
        pltpu.VMEM((SEL_PER_TILE, HID), _f32),
        pltpu.VMEM_SHARED((NP, HID), _f32),
    ],
)
def _agg_sel_kernel(ht_hbm, src_hbm, dst_hbm, zeros_hbm, dis_hbm, idx_hbm,
                    out_hbm, selacc_hbm, selht_hbm, seldis_hbm,
                    src_v, dst_v, *bufs):
    rows = bufs[:NBUF]
    gsems = bufs[NBUF:2 * NBUF]
    ssems = bufs[2 * NBUF:3 * NBUF]
    idxsel_v, selrow_v, acc_sp = bufs[3 * NBUF:]

    c, s, row0 = _agg_prologue(ht_hbm, src_hbm, dst_hbm, zeros_hbm,
                               src_v, dst_v, rows, gsems, ssems, acc_sp)
    pltpu.sync_copy(acc_sp.at[pl.ds(row0, ROWS_PER_TILE)],
                    out_hbm.at[c].at[pl.ds(row0, ROWS_PER_TILE)])

    base = s * SEL_PER_TILE
    pltpu.sync_copy(idx_hbm.at[pl.ds(base, SEL_PER_TILE)], idxsel_v)

    @pl.when(c == 0)
    def _():
        pltpu.async_copy(ht_hbm.at[idxsel_v], selrow_v, gsems[0]).wait()
        pltpu.sync_copy(selrow_v, selht_hbm.at[pl.ds(base, SEL_PER_TILE)])

    @pl.when(c == 1)
    def _():
        pltpu.async_copy(dis_hbm.at[idxsel_v], selrow_v, gsems[0]).wait()
        pltpu.sync_copy(selrow_v, seldis_hbm.at[pl.ds(base, SEL_PER_TILE)])

    plsc.subcore_barrier()
    pltpu.async_copy(out_hbm.at[c].at[idxsel_v], selrow_v, gsems[1]).wait()
    pltpu.sync_copy(selrow_v,
                    selacc_hbm.at[c].at[pl.ds(base, SEL_PER_TILE)])


MBLK = 1000


def _tcMM_body(x_ref, w0_ref, h_ref):
    h_ref[...] = jnp.dot(x_ref[...], w0_ref[...], preferred_element_type=_f32)


_tcMM = pl.pallas_call(
    _tcMM_body,
    grid=(N_NODES // (2 * MBLK),),
    in_specs=[
        pl.BlockSpec((MBLK, 2 * IN_CH), lambda i: (i, 0)),
        pl.BlockSpec((2 * IN_CH, 2 * HID), lambda i: (0, 0)),
    ],
    out_specs=pl.BlockSpec((MBLK, 2 * HID), lambda i: (i, 0)),
    out_shape=jax.ShapeDtypeStruct((NPH, 2 * HID), _f32),
)


def _tcScale_body(h_ref, dis_ref, ht_ref):
    ht_ref[...] = h_ref[...] * dis_ref[...]


_tcScale = pl.pallas_call(
    _tcScale_body,
    grid=(N_NODES // (2 * MBLK),),
    in_specs=[
        pl.BlockSpec((MBLK, 2 * HID), lambda i: (i, 0)),
        pl.BlockSpec((MBLK, 2 * HID), lambda i: (i, 0)),
    ],
    out_specs=pl.BlockSpec((MBLK, 2 * HID), lambda i: (i, 0)),
    out_shape=jax.ShapeDtypeStruct((NPH, 2 * HID), _f32),
)


def _tcB_body(accp_ref, ht0_ref, dis_ref, w1_ref, b0_ref, ht1_ref):
    dis = dis_ref[...]
    pre = (accp_ref[0] + accp_ref[1] + ht0_ref[...]) * dis + b0_ref[...]
    h1 = jnp.where(pre >= 0, pre, 0.01 * pre)
    ht1_ref[...] = jnp.dot(h1, w1_ref[...], preferred_element_type=_f32) * dis


_tcB = pl.pallas_call(
    _tcB_body,
    grid=(N_NODES // (2 * MBLK),),
    in_specs=[
        pl.BlockSpec((NC, MBLK, 2 * HID), lambda i: (0, i, 0)),
        pl.BlockSpec((MBLK, 2 * HID), lambda i: (i, 0)),
        pl.BlockSpec((MBLK, 2 * HID), lambda i: (i, 0)),
        pl.BlockSpec((2 * HID, 2 * HID), lambda i: (0, 0)),
        pl.BlockSpec((1, 2 * HID), lambda i: (0, 0)),
    ],
    out_specs=pl.BlockSpec((MBLK, 2 * HID), lambda i: (i, 0)),
    out_shape=jax.ShapeDtypeStruct((NPH, 2 * HID), _f32),
)


def _tcF_body(sacc_ref, sh_ref, sd_ref, b1_ref, wm_ref, bm_ref,
              hsel_ref, out_ref):
    pre = (sacc_ref[0] + sacc_ref[1] + sh_ref[...]) * sd_ref[...] + b1_ref[...]
    hsel = jnp.where(pre >= 0, pre, 0.01 * pre)
    hsel_ref[...] = hsel
    z = jnp.dot(hsel, wm_ref[...], preferred_element_type=_f32) + bm_ref[...]
    out_ref[...] = 1.0 / (1.0 + jnp.exp(-z))


_tcF = pl.pallas_call(
    _tcF_body,
    out_shape=[
        jax.ShapeDtypeStruct((N_IDX // 2, 2 * HID), _f32),
        jax.ShapeDtypeStruct((N_IDX // 2, 256), _f32),
    ],
)


def _blockdiag(w, rows, cols):
    z = jnp.zeros((2 * rows, 2 * cols), _f32)
    return z.at[:rows, :cols].set(w).at[rows:, cols:].set(w)


def kernel(x, edge_index, idx, W0, b0, W1, b1, Wm, bm):
    src2d = edge_index[0].reshape(NCH, CH)
    dst2d = edge_index[1].reshape(NCH, CH)
    x_p = x.reshape(N_NODES // 2, 2 * IN_CH)

    w0bd = _blockdiag(W0, IN_CH, HID)
    w1bd = _blockdiag(W1, HID, HID)
    wm_pad = jnp.pad(Wm, ((0, 0), (0, 128 - OUT_CH)))
    wmbd = _blockdiag(wm_pad, HID, 128)
    b0p = jnp.concatenate([b0, b0]).reshape(1, 2 * HID)
    b1p = jnp.concatenate([b1, b1]).reshape(1, 2 * HID)
    bm_pad = jnp.pad(bm, (0, 128 - OUT_CH))
    bmp = jnp.concatenate([bm_pad, bm_pad]).reshape(1, 256)

    h0_p = _tcMM(x_p, w0bd)
    dis64 = _dis_kernel(dst2d, _ONES_DEG, _ZEROS_DEG)
    dis_p = dis64.reshape(NPH, 2 * HID)
    ht0_p = _tcScale(h0_p, dis_p)
    acc0 = _agg_kernel(ht0_p.reshape(NP, HID), src2d, dst2d, _ZEROS_HID)
    ht1_p = _tcB(acc0.reshape(NC, NPH, 2 * HID), ht0_p, dis_p, w1bd, b0p)
    _, sel_acc, sel_ht, sel_dis = _agg_sel_kernel(
        ht1_p.reshape(NP, HID), src2d, dst2d, _ZEROS_HID, dis64, idx)
    hsel_p, out_pp = _tcF(sel_acc.reshape(NC, N_IDX // 2, 2 * HID),
                          sel_ht.reshape(N_IDX // 2, 2 * HID),
                          sel_dis.reshape(N_IDX // 2, 2 * HID),
                          b1p, wmbd, bmp)
    h_sel = hsel_p.reshape(N_IDX, HID)
    out = out_pp.reshape(N_IDX, 128)[:, :OUT_CH]
    return (h_sel, out)
